# baseline (speedup 1.0000x reference)
import functools

import jax
import jax.numpy as jnp
from jax import lax
from jax.experimental import pallas as pl
from jax.experimental.pallas import tpu as pltpu
from jax.experimental.pallas import tpu_sc as plsc

NC = 2
NS = 16
CH = 128

_SC_PARAMS = pltpu.CompilerParams(use_tc_tiling_on_sc=False)


def _sc_mesh():
    return plsc.VectorSubcoreMesh(
        core_axis_name="c", subcore_axis_name="s", num_cores=NC, num_subcores=NS
    )


def _make_edge_agg(NP, K, D, table_dtype):
    rows = NP // NS
    H = 2 if D >= 128 else 1
    assert K % (2 * H) == 0
    K2 = K // H

    @functools.partial(
        pl.kernel,
        out_type=jax.ShapeDtypeStruct((NC, NP, D), table_dtype),
        mesh=_sc_mesh(),
        compiler_params=_SC_PARAMS,
        scratch_types=[
            pltpu.VMEM((K2, CH), jnp.int32),
            pltpu.VMEM((K2, CH), jnp.int32),
            pltpu.VMEM((2, CH, D), table_dtype),
            pltpu.VMEM_SHARED((NP, D), table_dtype),
            pltpu.SemaphoreType.DMA((2,)),
        ],
    )
    def agg(src_hbm, dst_hbm, table_hbm, zeros_hbm, out_hbm, idx_s, idx_d, msg,
            acc, gsem):
        c = lax.axis_index("c")
        s = lax.axis_index("s")
        wid = c * NS + s
        pltpu.sync_copy(zeros_hbm.at[pl.ds(s * rows, rows)],
                        acc.at[pl.ds(s * rows, rows)])
        plsc.subcore_barrier()

        for h in range(H):
            pltpu.sync_copy(src_hbm.at[wid, pl.ds(h * K2, K2)], idx_s)
            pltpu.sync_copy(dst_hbm.at[wid, pl.ds(h * K2, K2)], idx_d)

            pltpu.async_copy(table_hbm.at[idx_s.at[0]], msg.at[0], gsem.at[0])

            @pl.loop(0, K2, step=2)
            def _(j):
                for b in range(2):
                    jj = j + b
                    pltpu.make_async_copy(
                        table_hbm.at[idx_s.at[jj]], msg.at[b], gsem.at[b]).wait()
                    nb = 1 - b

                    @pl.when(jj + 1 < K2)
                    def _():
                        pltpu.async_copy(
                            table_hbm.at[idx_s.at[jj + 1]], msg.at[nb],
                            gsem.at[nb])

                    pltpu.sync_copy(msg.at[b], acc.at[idx_d.at[jj]], add=True)

        plsc.subcore_barrier()
        pltpu.sync_copy(acc.at[pl.ds(s * rows, rows)],
                        out_hbm.at[c, pl.ds(s * rows, rows)])

    return agg


def _make_deg(NP, K):
    D = 16
    rows = NP // NS

    @functools.partial(
        pl.kernel,
        out_type=jax.ShapeDtypeStruct((NC, NP, D), jnp.float32),
        mesh=_sc_mesh(),
        compiler_params=_SC_PARAMS,
        scratch_types=[
            pltpu.VMEM((K, CH), jnp.int32),
            pltpu.VMEM((CH, D), jnp.float32),
            pltpu.VMEM_SHARED((NP, D), jnp.float32),
            pltpu.SemaphoreType.DMA,
        ],
    )
    def deg(dst_hbm, ones_hbm, zeros_hbm, out_hbm, idx_d, ones_v, acc, ssem):
        c = lax.axis_index("c")
        s = lax.axis_index("s")
        wid = c * NS + s
        pltpu.sync_copy(dst_hbm.at[wid], idx_d)
        pltpu.sync_copy(ones_hbm, ones_v)
        pltpu.sync_copy(zeros_hbm.at[pl.ds(s * rows, rows)],
                        acc.at[pl.ds(s * rows, rows)])
        plsc.subcore_barrier()

        @pl.loop(0, K)
        def _(j):
            pltpu.async_copy(ones_v, acc.at[idx_d.at[j]], ssem, add=True)

        @pl.loop(0, K)
        def _(j):
            pltpu.make_async_copy(ones_v, acc.at[idx_d.at[j]], ssem).wait()

        plsc.subcore_barrier()
        pltpu.sync_copy(acc.at[pl.ds(s * rows, rows)],
                        out_hbm.at[c, pl.ds(s * rows, rows)])

    return deg


def _tc_matmul(x, w, BM):
    NP, Din = x.shape
    Dout = w.shape[1]

    def body(x_ref, w_ref, o_ref):
        o_ref[...] = jnp.dot(x_ref[...], w_ref[...],
                             preferred_element_type=jnp.float32)

    return pl.pallas_call(
        body,
        grid=(NP // BM,),
        in_specs=[
            pl.BlockSpec((BM, Din), lambda i: (i, 0)),
            pl.BlockSpec((Din, Dout), lambda i: (0, 0)),
        ],
        out_specs=pl.BlockSpec((BM, Dout), lambda i: (i, 0)),
        out_shape=jax.ShapeDtypeStruct((NP, Dout), jnp.float32),
    )(x, w)


def _tc_norm_scale(dpart, xw, BM):
    NP, D = xw.shape

    def body(dp_ref, xw_ref, y1_ref, dinv_ref):
        deg = dp_ref[0, :, 0:1] + dp_ref[1, :, 0:1] + 1.0
        dinv = lax.rsqrt(deg)
        dinv_ref[...] = dinv
        y1_ref[...] = xw_ref[...] * dinv

    return pl.pallas_call(
        body,
        grid=(NP // BM,),
        in_specs=[
            pl.BlockSpec((2, BM, 16), lambda i: (0, i, 0)),
            pl.BlockSpec((BM, D), lambda i: (i, 0)),
        ],
        out_specs=[
            pl.BlockSpec((BM, D), lambda i: (i, 0)),
            pl.BlockSpec((BM, 1), lambda i: (i, 0)),
        ],
        out_shape=[
            jax.ShapeDtypeStruct((NP, D), jnp.float32),
            jax.ShapeDtypeStruct((NP, 1), jnp.float32),
        ],
    )(dpart, xw)


def _tc_mid(p, y1, dinv, b1, w2p, BM):
    NP, D = y1.shape
    D2 = w2p.shape[1]

    def body(p_ref, y1_ref, dinv_ref, b1_ref, w2_ref, y2_ref):
        agg = p_ref[0] + p_ref[1] + y1_ref[...]
        h = jnp.maximum(dinv_ref[...] * agg + b1_ref[...], 0.0)
        z = jnp.dot(h, w2_ref[...], preferred_element_type=jnp.float32)
        y2_ref[...] = dinv_ref[...] * z

    return pl.pallas_call(
        body,
        grid=(NP // BM,),
        in_specs=[
            pl.BlockSpec((2, BM, D), lambda i: (0, i, 0)),
            pl.BlockSpec((BM, D), lambda i: (i, 0)),
            pl.BlockSpec((BM, 1), lambda i: (i, 0)),
            pl.BlockSpec((1, D), lambda i: (0, 0)),
            pl.BlockSpec((D, D2), lambda i: (0, 0)),
        ],
        out_specs=pl.BlockSpec((BM, D2), lambda i: (i, 0)),
        out_shape=jax.ShapeDtypeStruct((NP, D2), jnp.float32),
    )(p, y1, dinv, b1, w2p)


def _tc_final(q, y2, dinv, b2p, n_valid, BM):
    NP, D2 = y2.shape

    def body(q_ref, y2_ref, dinv_ref, b2_ref, o_ref):
        o = dinv_ref[...] * (q_ref[0] + q_ref[1] + y2_ref[...]) + b2_ref[...]
        col = lax.broadcasted_iota(jnp.int32, (BM, D2), 1)
        valid = col < n_valid
        neg = jnp.where(valid, o, -jnp.inf)
        m = jnp.max(neg, axis=1, keepdims=True)
        e = jnp.where(valid, jnp.exp(o - m), 0.0)
        lse = m + jnp.log(jnp.sum(e, axis=1, keepdims=True))
        o_ref[...] = o - lse

    return pl.pallas_call(
        body,
        grid=(NP // BM,),
        in_specs=[
            pl.BlockSpec((2, BM, D2), lambda i: (0, i, 0)),
            pl.BlockSpec((BM, D2), lambda i: (i, 0)),
            pl.BlockSpec((BM, 1), lambda i: (i, 0)),
            pl.BlockSpec((1, D2), lambda i: (0, 0)),
        ],
        out_specs=pl.BlockSpec((BM, D2), lambda i: (i, 0)),
        out_shape=jax.ShapeDtypeStruct((NP, D2), jnp.float32),
    )(q, y2, dinv, b2p)


def kernel(x, edge_index, W1, b1, W2, b2):
    N, D_IN = x.shape
    E = edge_index.shape[1]
    D_HID = W1.shape[1]
    D_OUT = W2.shape[1]
    D2 = 16

    BM = 1280
    NP = ((N + 1 + BM - 1) // BM) * BM
    JUNK = N

    W = NC * NS
    K = (E + W * CH - 1) // (W * CH)
    K = ((K + 3) // 4) * 4
    EP = W * K * CH

    src = edge_index[0]
    dst = edge_index[1]
    pad = jnp.full((EP - E,), JUNK, dtype=jnp.int32)
    src_r = jnp.concatenate([src.astype(jnp.int32), pad]).reshape(W, K, CH)
    dst_r = jnp.concatenate([dst.astype(jnp.int32), pad]).reshape(W, K, CH)
    x_pad = jnp.zeros((NP, D_IN), jnp.float32).at[:N].set(x)
    zeros_hid = jnp.zeros((NP, D_HID), jnp.float32)
    zeros_16 = jnp.zeros((NP, D2), jnp.float32)
    ones_blk = jnp.ones((CH, D2), jnp.float32)
    w2p = jnp.zeros((D_HID, D2), jnp.float32).at[:, :D_OUT].set(W2)
    b1r = b1.reshape(1, D_HID)
    b2p = jnp.zeros((1, D2), jnp.float32).at[0, :D_OUT].set(b2)

    dpart = _make_deg(NP, K)(dst_r, ones_blk, zeros_16)
    xw = _tc_matmul(x_pad, W1, BM)

    y1, dinv = _tc_norm_scale(dpart, xw, BM)

    p = _make_edge_agg(NP, K, D_HID, jnp.float32)(src_r, dst_r, y1, zeros_hid)

    y2 = _tc_mid(p, y1, dinv, b1r, w2p, BM)

    q = _make_edge_agg(NP, K, D2, jnp.float32)(src_r, dst_r, y2, zeros_16)

    o = _tc_final(q, y2, dinv, b2p, D_OUT, BM)
    return o[:N, :D_OUT]


# speedup vs baseline: 2.0532x; 2.0532x over previous
import functools

import jax
import jax.numpy as jnp
from jax import lax
from jax.experimental import pallas as pl
from jax.experimental.pallas import tpu as pltpu
from jax.experimental.pallas import tpu_sc as plsc

NC = 2
NS = 16
CH = 128

_SC_PARAMS = pltpu.CompilerParams(use_tc_tiling_on_sc=False)


def _sc_mesh():
    return plsc.VectorSubcoreMesh(
        core_axis_name="c", subcore_axis_name="s", num_cores=NC, num_subcores=NS
    )


def _make_edge_agg(NP, K, D, table_dtype):
    rows = NP // NS
    H = 2 if D >= 128 else 1
    assert K % (2 * H) == 0
    K2 = K // H

    @functools.partial(
        pl.kernel,
        out_type=jax.ShapeDtypeStruct((NC, NP, D), table_dtype),
        mesh=_sc_mesh(),
        compiler_params=_SC_PARAMS,
        scratch_types=[
            pltpu.VMEM((K2, CH), jnp.int32),
            pltpu.VMEM((K2, CH), jnp.int32),
            pltpu.VMEM((2, CH, D), table_dtype),
            pltpu.VMEM_SHARED((NP, D), table_dtype),
            pltpu.SemaphoreType.DMA((2,)),
        ],
    )
    def agg(src_hbm, dst_hbm, table_hbm, zeros_hbm, out_hbm, idx_s, idx_d, msg,
            acc, gsem):
        c = lax.axis_index("c")
        s = lax.axis_index("s")
        wid = c * NS + s
        pltpu.sync_copy(zeros_hbm.at[pl.ds(s * rows, rows)],
                        acc.at[pl.ds(s * rows, rows)])
        plsc.subcore_barrier()

        for h in range(H):
            pltpu.sync_copy(src_hbm.at[wid, pl.ds(h * K2, K2)], idx_s)
            pltpu.sync_copy(dst_hbm.at[wid, pl.ds(h * K2, K2)], idx_d)

            pltpu.async_copy(table_hbm.at[idx_s.at[0]], msg.at[0], gsem.at[0])

            @pl.loop(0, K2, step=2)
            def _(j):
                for b in range(2):
                    jj = j + b
                    pltpu.make_async_copy(
                        table_hbm.at[idx_s.at[jj]], msg.at[b], gsem.at[b]).wait()
                    nb = 1 - b

                    @pl.when(jj + 1 < K2)
                    def _():
                        pltpu.async_copy(
                            table_hbm.at[idx_s.at[jj + 1]], msg.at[nb],
                            gsem.at[nb])

                    pltpu.sync_copy(msg.at[b], acc.at[idx_d.at[jj]], add=True)

        plsc.subcore_barrier()
        pltpu.sync_copy(acc.at[pl.ds(s * rows, rows)],
                        out_hbm.at[c, pl.ds(s * rows, rows)])

    return agg


def _make_deg(NP, K):
    D = 16
    rows = NP // NS

    @functools.partial(
        pl.kernel,
        out_type=jax.ShapeDtypeStruct((NC, NP, D), jnp.float32),
        mesh=_sc_mesh(),
        compiler_params=_SC_PARAMS,
        scratch_types=[
            pltpu.VMEM((K, CH), jnp.int32),
            pltpu.VMEM((CH, D), jnp.float32),
            pltpu.VMEM_SHARED((NP, D), jnp.float32),
            pltpu.SemaphoreType.DMA,
        ],
    )
    def deg(dst_hbm, ones_hbm, zeros_hbm, out_hbm, idx_d, ones_v, acc, ssem):
        c = lax.axis_index("c")
        s = lax.axis_index("s")
        wid = c * NS + s
        pltpu.sync_copy(dst_hbm.at[wid], idx_d)
        pltpu.sync_copy(ones_hbm, ones_v)
        pltpu.sync_copy(zeros_hbm.at[pl.ds(s * rows, rows)],
                        acc.at[pl.ds(s * rows, rows)])
        plsc.subcore_barrier()

        @pl.loop(0, K)
        def _(j):
            pltpu.async_copy(ones_v, acc.at[idx_d.at[j]], ssem, add=True)

        @pl.loop(0, K)
        def _(j):
            pltpu.make_async_copy(ones_v, acc.at[idx_d.at[j]], ssem).wait()

        plsc.subcore_barrier()
        pltpu.sync_copy(acc.at[pl.ds(s * rows, rows)],
                        out_hbm.at[c, pl.ds(s * rows, rows)])

    return deg


def _tc_matmul(x, w, BM):
    NP, Din = x.shape
    Dout = w.shape[1]

    def body(x_ref, w_ref, o_ref):
        o_ref[...] = jnp.dot(x_ref[...], w_ref[...],
                             preferred_element_type=jnp.float32)

    return pl.pallas_call(
        body,
        grid=(NP // BM,),
        in_specs=[
            pl.BlockSpec((BM, Din), lambda i: (i, 0)),
            pl.BlockSpec((Din, Dout), lambda i: (0, 0)),
        ],
        out_specs=pl.BlockSpec((BM, Dout), lambda i: (i, 0)),
        out_shape=jax.ShapeDtypeStruct((NP, Dout), jnp.float32),
    )(x, w)


def _tc_norm_scale(dpart, xw, BM):
    NP, D = xw.shape

    def body(dp_ref, xw_ref, y1_ref, dinv_ref):
        deg = dp_ref[0, :, 0:1] + dp_ref[1, :, 0:1] + 1.0
        dinv = lax.rsqrt(deg)
        dinv_ref[...] = dinv
        y1_ref[...] = xw_ref[...] * dinv

    return pl.pallas_call(
        body,
        grid=(NP // BM,),
        in_specs=[
            pl.BlockSpec((2, BM, 16), lambda i: (0, i, 0)),
            pl.BlockSpec((BM, D), lambda i: (i, 0)),
        ],
        out_specs=[
            pl.BlockSpec((BM, D), lambda i: (i, 0)),
            pl.BlockSpec((BM, 1), lambda i: (i, 0)),
        ],
        out_shape=[
            jax.ShapeDtypeStruct((NP, D), jnp.float32),
            jax.ShapeDtypeStruct((NP, 1), jnp.float32),
        ],
    )(dpart, xw)


def _tc_mid(p, y1, dinv, b1, w2p, BM):
    NP, D = y1.shape
    D2 = w2p.shape[1]

    def body(p_ref, y1_ref, dinv_ref, b1_ref, w2_ref, y2_ref):
        agg = p_ref[0] + p_ref[1] + y1_ref[...]
        h = jnp.maximum(dinv_ref[...] * agg + b1_ref[...], 0.0)
        z = jnp.dot(h, w2_ref[...], preferred_element_type=jnp.float32)
        y2_ref[...] = dinv_ref[...] * z

    return pl.pallas_call(
        body,
        grid=(NP // BM,),
        in_specs=[
            pl.BlockSpec((2, BM, D), lambda i: (0, i, 0)),
            pl.BlockSpec((BM, D), lambda i: (i, 0)),
            pl.BlockSpec((BM, 1), lambda i: (i, 0)),
            pl.BlockSpec((1, D), lambda i: (0, 0)),
            pl.BlockSpec((D, D2), lambda i: (0, 0)),
        ],
        out_specs=pl.BlockSpec((BM, D2), lambda i: (i, 0)),
        out_shape=jax.ShapeDtypeStruct((NP, D2), jnp.float32),
    )(p, y1, dinv, b1, w2p)


def _tc_final(q, y2, dinv, b2p, n_valid, BM):
    NP, D2 = y2.shape

    def body(q_ref, y2_ref, dinv_ref, b2_ref, o_ref):
        o = dinv_ref[...] * (q_ref[0] + q_ref[1] + y2_ref[...]) + b2_ref[...]
        col = lax.broadcasted_iota(jnp.int32, (BM, D2), 1)
        valid = col < n_valid
        neg = jnp.where(valid, o, -jnp.inf)
        m = jnp.max(neg, axis=1, keepdims=True)
        e = jnp.where(valid, jnp.exp(o - m), 0.0)
        lse = m + jnp.log(jnp.sum(e, axis=1, keepdims=True))
        o_ref[...] = o - lse

    return pl.pallas_call(
        body,
        grid=(NP // BM,),
        in_specs=[
            pl.BlockSpec((2, BM, D2), lambda i: (0, i, 0)),
            pl.BlockSpec((BM, D2), lambda i: (i, 0)),
            pl.BlockSpec((BM, 1), lambda i: (i, 0)),
            pl.BlockSpec((1, D2), lambda i: (0, 0)),
        ],
        out_specs=pl.BlockSpec((BM, D2), lambda i: (i, 0)),
        out_shape=jax.ShapeDtypeStruct((NP, D2), jnp.float32),
    )(q, y2, dinv, b2p)


def kernel(x, edge_index, W1, b1, W2, b2):
    N, D_IN = x.shape
    E = edge_index.shape[1]
    D_HID = W1.shape[1]
    D_OUT = W2.shape[1]
    D2 = 16

    BM = 1280
    NP = ((N + 1 + BM - 1) // BM) * BM
    JUNK = N

    W = NC * NS
    K = (E + W * CH - 1) // (W * CH)
    K = ((K + 3) // 4) * 4
    EP = W * K * CH

    src = edge_index[0]
    dst = edge_index[1]
    pad = JUNK + (jnp.arange(EP - E, dtype=jnp.int32) % (NP - N))
    src_r = jnp.concatenate([src.astype(jnp.int32), pad]).reshape(W, K, CH)
    dst_r = jnp.concatenate([dst.astype(jnp.int32), pad]).reshape(W, K, CH)
    x_pad = jnp.zeros((NP, D_IN), jnp.float32).at[:N].set(x)
    zeros_hid = jnp.zeros((NP, D_HID), jnp.float32)
    zeros_16 = jnp.zeros((NP, D2), jnp.float32)
    ones_blk = jnp.ones((CH, D2), jnp.float32)
    w2p = jnp.zeros((D_HID, D2), jnp.float32).at[:, :D_OUT].set(W2)
    b1r = b1.reshape(1, D_HID)
    b2p = jnp.zeros((1, D2), jnp.float32).at[0, :D_OUT].set(b2)

    dpart = _make_deg(NP, K)(dst_r, ones_blk, zeros_16)
    xw = _tc_matmul(x_pad, W1, BM)

    y1, dinv = _tc_norm_scale(dpart, xw, BM)

    p = _make_edge_agg(NP, K, D_HID, jnp.float32)(src_r, dst_r, y1, zeros_hid)

    y2 = _tc_mid(p, y1, dinv, b1r, w2p, BM)

    q = _make_edge_agg(NP, K, D2, jnp.float32)(src_r, dst_r, y2, zeros_16)

    o = _tc_final(q, y2, dinv, b2p, D_OUT, BM)
    return o[:N, :D_OUT]


# speedup vs baseline: 2.3019x; 1.1211x over previous
import functools

import jax
import jax.numpy as jnp
from jax import lax
from jax.experimental import pallas as pl
from jax.experimental.pallas import tpu as pltpu
from jax.experimental.pallas import tpu_sc as plsc

NC = 2
NS = 16
CH = 128

_SC_PARAMS = pltpu.CompilerParams(use_tc_tiling_on_sc=False)


def _sc_mesh():
    return plsc.VectorSubcoreMesh(
        core_axis_name="c", subcore_axis_name="s", num_cores=NC, num_subcores=NS
    )


def _make_edge_agg(NP, K, D, table_dtype):
    rows = NP // NS
    H = 2 if D >= 128 else 1
    NB = 2 if D >= 128 else 4
    assert K % (2 * H) == 0 and K % NB == 0
    K2 = K // H

    @functools.partial(
        pl.kernel,
        out_type=jax.ShapeDtypeStruct((NC, NP, D), table_dtype),
        mesh=_sc_mesh(),
        compiler_params=_SC_PARAMS,
        scratch_types=[
            pltpu.VMEM((K2, CH), jnp.int32),
            pltpu.VMEM((K2, CH), jnp.int32),
            pltpu.VMEM((NB, CH, D), table_dtype),
            pltpu.VMEM_SHARED((NP, D), table_dtype),
            pltpu.SemaphoreType.DMA((NB,)),
            pltpu.SemaphoreType.DMA((NB,)),
        ],
    )
    def agg(src_hbm, dst_hbm, table_hbm, zeros_hbm, out_hbm, idx_s, idx_d, msg,
            acc, gsem, ssem):
        c = lax.axis_index("c")
        s = lax.axis_index("s")
        wid = c * NS + s
        pltpu.sync_copy(zeros_hbm.at[pl.ds(s * rows, rows)],
                        acc.at[pl.ds(s * rows, rows)])
        plsc.subcore_barrier()

        if NB == 2:
            for h in range(H):
                pltpu.sync_copy(src_hbm.at[wid, pl.ds(h * K2, K2)], idx_s)
                pltpu.sync_copy(dst_hbm.at[wid, pl.ds(h * K2, K2)], idx_d)

                pltpu.async_copy(table_hbm.at[idx_s.at[0]], msg.at[0],
                                 gsem.at[0])

                @pl.loop(0, K2, step=2)
                def _(j):
                    for b in range(2):
                        jj = j + b
                        pltpu.make_async_copy(
                            table_hbm.at[idx_s.at[jj]], msg.at[b],
                            gsem.at[b]).wait()
                        nb = 1 - b

                        @pl.when(jj + 1 < K2)
                        def _():
                            pltpu.async_copy(
                                table_hbm.at[idx_s.at[jj + 1]], msg.at[nb],
                                gsem.at[nb])

                        pltpu.sync_copy(msg.at[b], acc.at[idx_d.at[jj]],
                                        add=True)
        else:
            pltpu.sync_copy(src_hbm.at[wid], idx_s)
            pltpu.sync_copy(dst_hbm.at[wid], idx_d)

            @pl.loop(0, K, step=NB)
            def _(j):
                for b in range(NB):
                    pltpu.async_copy(table_hbm.at[idx_s.at[j + b]], msg.at[b],
                                     gsem.at[b])
                for b in range(NB):
                    pltpu.make_async_copy(
                        table_hbm.at[idx_s.at[j + b]], msg.at[b],
                        gsem.at[b]).wait()
                    pltpu.async_copy(msg.at[b], acc.at[idx_d.at[j + b]],
                                     ssem.at[b], add=True)
                for b in range(NB):
                    pltpu.make_async_copy(
                        msg.at[b], acc.at[idx_d.at[j + b]],
                        ssem.at[b]).wait()

        plsc.subcore_barrier()
        pltpu.sync_copy(acc.at[pl.ds(s * rows, rows)],
                        out_hbm.at[c, pl.ds(s * rows, rows)])

    return agg


def _make_deg(NP, K):
    D = 16
    rows = NP // NS

    @functools.partial(
        pl.kernel,
        out_type=jax.ShapeDtypeStruct((NC, NP, D), jnp.float32),
        mesh=_sc_mesh(),
        compiler_params=_SC_PARAMS,
        scratch_types=[
            pltpu.VMEM((K, CH), jnp.int32),
            pltpu.VMEM((CH, D), jnp.float32),
            pltpu.VMEM_SHARED((NP, D), jnp.float32),
            pltpu.SemaphoreType.DMA,
        ],
    )
    def deg(dst_hbm, ones_hbm, zeros_hbm, out_hbm, idx_d, ones_v, acc, ssem):
        c = lax.axis_index("c")
        s = lax.axis_index("s")
        wid = c * NS + s
        pltpu.sync_copy(dst_hbm.at[wid], idx_d)
        pltpu.sync_copy(ones_hbm, ones_v)
        pltpu.sync_copy(zeros_hbm.at[pl.ds(s * rows, rows)],
                        acc.at[pl.ds(s * rows, rows)])
        plsc.subcore_barrier()

        @pl.loop(0, K)
        def _(j):
            pltpu.async_copy(ones_v, acc.at[idx_d.at[j]], ssem, add=True)

        @pl.loop(0, K)
        def _(j):
            pltpu.make_async_copy(ones_v, acc.at[idx_d.at[j]], ssem).wait()

        plsc.subcore_barrier()
        pltpu.sync_copy(acc.at[pl.ds(s * rows, rows)],
                        out_hbm.at[c, pl.ds(s * rows, rows)])

    return deg


def _tc_matmul(x, w, BM):
    NP, Din = x.shape
    Dout = w.shape[1]

    def body(x_ref, w_ref, o_ref):
        o_ref[...] = jnp.dot(x_ref[...], w_ref[...],
                             preferred_element_type=jnp.float32)

    return pl.pallas_call(
        body,
        grid=(NP // BM,),
        in_specs=[
            pl.BlockSpec((BM, Din), lambda i: (i, 0)),
            pl.BlockSpec((Din, Dout), lambda i: (0, 0)),
        ],
        out_specs=pl.BlockSpec((BM, Dout), lambda i: (i, 0)),
        out_shape=jax.ShapeDtypeStruct((NP, Dout), jnp.float32),
    )(x, w)


def _tc_norm_scale(dpart, xw, BM):
    NP, D = xw.shape

    def body(dp_ref, xw_ref, y1_ref, dinv_ref):
        deg = dp_ref[0, :, 0:1] + dp_ref[1, :, 0:1] + 1.0
        dinv = lax.rsqrt(deg)
        dinv_ref[...] = dinv
        y1_ref[...] = xw_ref[...] * dinv

    return pl.pallas_call(
        body,
        grid=(NP // BM,),
        in_specs=[
            pl.BlockSpec((2, BM, 16), lambda i: (0, i, 0)),
            pl.BlockSpec((BM, D), lambda i: (i, 0)),
        ],
        out_specs=[
            pl.BlockSpec((BM, D), lambda i: (i, 0)),
            pl.BlockSpec((BM, 1), lambda i: (i, 0)),
        ],
        out_shape=[
            jax.ShapeDtypeStruct((NP, D), jnp.float32),
            jax.ShapeDtypeStruct((NP, 1), jnp.float32),
        ],
    )(dpart, xw)


def _tc_mid(p, y1, dinv, b1, w2p, BM):
    NP, D = y1.shape
    D2 = w2p.shape[1]

    def body(p_ref, y1_ref, dinv_ref, b1_ref, w2_ref, y2_ref):
        agg = p_ref[0] + p_ref[1] + y1_ref[...]
        h = jnp.maximum(dinv_ref[...] * agg + b1_ref[...], 0.0)
        z = jnp.dot(h, w2_ref[...], preferred_element_type=jnp.float32)
        y2_ref[...] = dinv_ref[...] * z

    return pl.pallas_call(
        body,
        grid=(NP // BM,),
        in_specs=[
            pl.BlockSpec((2, BM, D), lambda i: (0, i, 0)),
            pl.BlockSpec((BM, D), lambda i: (i, 0)),
            pl.BlockSpec((BM, 1), lambda i: (i, 0)),
            pl.BlockSpec((1, D), lambda i: (0, 0)),
            pl.BlockSpec((D, D2), lambda i: (0, 0)),
        ],
        out_specs=pl.BlockSpec((BM, D2), lambda i: (i, 0)),
        out_shape=jax.ShapeDtypeStruct((NP, D2), jnp.float32),
    )(p, y1, dinv, b1, w2p)


def _tc_final(q, y2, dinv, b2p, n_valid, BM):
    NP, D2 = y2.shape

    def body(q_ref, y2_ref, dinv_ref, b2_ref, o_ref):
        o = dinv_ref[...] * (q_ref[0] + q_ref[1] + y2_ref[...]) + b2_ref[...]
        col = lax.broadcasted_iota(jnp.int32, (BM, D2), 1)
        valid = col < n_valid
        neg = jnp.where(valid, o, -jnp.inf)
        m = jnp.max(neg, axis=1, keepdims=True)
        e = jnp.where(valid, jnp.exp(o - m), 0.0)
        lse = m + jnp.log(jnp.sum(e, axis=1, keepdims=True))
        o_ref[...] = o - lse

    return pl.pallas_call(
        body,
        grid=(NP // BM,),
        in_specs=[
            pl.BlockSpec((2, BM, D2), lambda i: (0, i, 0)),
            pl.BlockSpec((BM, D2), lambda i: (i, 0)),
            pl.BlockSpec((BM, 1), lambda i: (i, 0)),
            pl.BlockSpec((1, D2), lambda i: (0, 0)),
        ],
        out_specs=pl.BlockSpec((BM, D2), lambda i: (i, 0)),
        out_shape=jax.ShapeDtypeStruct((NP, D2), jnp.float32),
    )(q, y2, dinv, b2p)


def kernel(x, edge_index, W1, b1, W2, b2):
    N, D_IN = x.shape
    E = edge_index.shape[1]
    D_HID = W1.shape[1]
    D_OUT = W2.shape[1]
    D2 = 16

    BM = 1280
    NP = ((N + 1 + BM - 1) // BM) * BM
    JUNK = N

    W = NC * NS
    K = (E + W * CH - 1) // (W * CH)
    K = ((K + 15) // 16) * 16
    EP = W * K * CH

    src = edge_index[0]
    dst = edge_index[1]
    pad = JUNK + (jnp.arange(EP - E, dtype=jnp.int32) % (NP - N))
    src_r = jnp.concatenate([src.astype(jnp.int32), pad]).reshape(W, K, CH)
    dst_r = jnp.concatenate([dst.astype(jnp.int32), pad]).reshape(W, K, CH)
    x_pad = jnp.zeros((NP, D_IN), jnp.float32).at[:N].set(x)
    zeros_hid = jnp.zeros((NP, D_HID), jnp.float32)
    zeros_16 = jnp.zeros((NP, D2), jnp.float32)
    ones_blk = jnp.ones((CH, D2), jnp.float32)
    w2p = jnp.zeros((D_HID, D2), jnp.float32).at[:, :D_OUT].set(W2)
    b1r = b1.reshape(1, D_HID)
    b2p = jnp.zeros((1, D2), jnp.float32).at[0, :D_OUT].set(b2)

    dpart = _make_deg(NP, K)(dst_r, ones_blk, zeros_16)
    xw = _tc_matmul(x_pad, W1, BM)

    y1, dinv = _tc_norm_scale(dpart, xw, BM)

    p = _make_edge_agg(NP, K, D_HID, jnp.float32)(src_r, dst_r, y1, zeros_hid)

    y2 = _tc_mid(p, y1, dinv, b1r, w2p, BM)

    q = _make_edge_agg(NP, K, D2, jnp.float32)(src_r, dst_r, y2, zeros_16)

    o = _tc_final(q, y2, dinv, b2p, D_OUT, BM)
    return o[:N, :D_OUT]


# speedup vs baseline: 2.3675x; 1.0285x over previous
import functools

import jax
import jax.numpy as jnp
from jax import lax
from jax.experimental import pallas as pl
from jax.experimental.pallas import tpu as pltpu
from jax.experimental.pallas import tpu_sc as plsc

NC = 2
NS = 16
CH = 128

_SC_PARAMS = pltpu.CompilerParams(use_tc_tiling_on_sc=False)


def _sc_mesh():
    return plsc.VectorSubcoreMesh(
        core_axis_name="c", subcore_axis_name="s", num_cores=NC, num_subcores=NS
    )


def _make_edge_agg(NP, K, D, table_dtype):
    rows = NP // NS
    H = 2 if D >= 128 else 1
    NB = 2 if D >= 128 else 8
    assert K % (2 * H) == 0 and K % NB == 0
    K2 = K // H

    @functools.partial(
        pl.kernel,
        out_type=jax.ShapeDtypeStruct((NC, NP, D), table_dtype),
        mesh=_sc_mesh(),
        compiler_params=_SC_PARAMS,
        scratch_types=[
            pltpu.VMEM((K2, CH), jnp.int32),
            pltpu.VMEM((K2, CH), jnp.int32),
            pltpu.VMEM((NB, CH, D), table_dtype),
            pltpu.VMEM_SHARED((NP, D), table_dtype),
            pltpu.SemaphoreType.DMA((NB,)),
            pltpu.SemaphoreType.DMA((NB,)),
        ],
    )
    def agg(src_hbm, dst_hbm, table_hbm, zeros_hbm, out_hbm, idx_s, idx_d, msg,
            acc, gsem, ssem):
        c = lax.axis_index("c")
        s = lax.axis_index("s")
        wid = c * NS + s
        pltpu.sync_copy(zeros_hbm.at[pl.ds(s * rows, rows)],
                        acc.at[pl.ds(s * rows, rows)])
        plsc.subcore_barrier()

        if NB == 2:
            for h in range(H):
                pltpu.sync_copy(src_hbm.at[wid, pl.ds(h * K2, K2)], idx_s)
                pltpu.sync_copy(dst_hbm.at[wid, pl.ds(h * K2, K2)], idx_d)

                pltpu.async_copy(table_hbm.at[idx_s.at[0]], msg.at[0],
                                 gsem.at[0])

                @pl.loop(0, K2, step=2)
                def _(j):
                    for b in range(2):
                        jj = j + b
                        pltpu.make_async_copy(
                            table_hbm.at[idx_s.at[jj]], msg.at[b],
                            gsem.at[b]).wait()
                        nb = 1 - b

                        @pl.when(jj + 1 < K2)
                        def _():
                            pltpu.async_copy(
                                table_hbm.at[idx_s.at[jj + 1]], msg.at[nb],
                                gsem.at[nb])

                        pltpu.sync_copy(msg.at[b], acc.at[idx_d.at[jj]],
                                        add=True)
        else:
            pltpu.sync_copy(src_hbm.at[wid], idx_s)
            pltpu.sync_copy(dst_hbm.at[wid], idx_d)

            @pl.loop(0, K, step=NB)
            def _(j):
                for b in range(NB):
                    pltpu.async_copy(table_hbm.at[idx_s.at[j + b]], msg.at[b],
                                     gsem.at[b])
                for b in range(NB):
                    pltpu.make_async_copy(
                        table_hbm.at[idx_s.at[j + b]], msg.at[b],
                        gsem.at[b]).wait()
                    pltpu.async_copy(msg.at[b], acc.at[idx_d.at[j + b]],
                                     ssem.at[b], add=True)
                for b in range(NB):
                    pltpu.make_async_copy(
                        msg.at[b], acc.at[idx_d.at[j + b]],
                        ssem.at[b]).wait()

        plsc.subcore_barrier()
        pltpu.sync_copy(acc.at[pl.ds(s * rows, rows)],
                        out_hbm.at[c, pl.ds(s * rows, rows)])

    return agg


def _make_deg(NP, K):
    D = 16
    rows = NP // NS

    @functools.partial(
        pl.kernel,
        out_type=jax.ShapeDtypeStruct((NC, NP, D), jnp.float32),
        mesh=_sc_mesh(),
        compiler_params=_SC_PARAMS,
        scratch_types=[
            pltpu.VMEM((K, CH), jnp.int32),
            pltpu.VMEM((CH, D), jnp.float32),
            pltpu.VMEM_SHARED((NP, D), jnp.float32),
            pltpu.SemaphoreType.DMA,
        ],
    )
    def deg(dst_hbm, ones_hbm, zeros_hbm, out_hbm, idx_d, ones_v, acc, ssem):
        c = lax.axis_index("c")
        s = lax.axis_index("s")
        wid = c * NS + s
        pltpu.sync_copy(dst_hbm.at[wid], idx_d)
        pltpu.sync_copy(ones_hbm, ones_v)
        pltpu.sync_copy(zeros_hbm.at[pl.ds(s * rows, rows)],
                        acc.at[pl.ds(s * rows, rows)])
        plsc.subcore_barrier()

        @pl.loop(0, K)
        def _(j):
            pltpu.async_copy(ones_v, acc.at[idx_d.at[j]], ssem, add=True)

        @pl.loop(0, K)
        def _(j):
            pltpu.make_async_copy(ones_v, acc.at[idx_d.at[j]], ssem).wait()

        plsc.subcore_barrier()
        pltpu.sync_copy(acc.at[pl.ds(s * rows, rows)],
                        out_hbm.at[c, pl.ds(s * rows, rows)])

    return deg


def _tc_matmul(x, w, BM):
    NP, Din = x.shape
    Dout = w.shape[1]

    def body(x_ref, w_ref, o_ref):
        o_ref[...] = jnp.dot(x_ref[...], w_ref[...],
                             preferred_element_type=jnp.float32)

    return pl.pallas_call(
        body,
        grid=(NP // BM,),
        in_specs=[
            pl.BlockSpec((BM, Din), lambda i: (i, 0)),
            pl.BlockSpec((Din, Dout), lambda i: (0, 0)),
        ],
        out_specs=pl.BlockSpec((BM, Dout), lambda i: (i, 0)),
        out_shape=jax.ShapeDtypeStruct((NP, Dout), jnp.float32),
    )(x, w)


def _tc_norm_scale(dpart, xw, BM):
    NP, D = xw.shape

    def body(dp_ref, xw_ref, y1_ref, dinv_ref):
        deg = dp_ref[0, :, 0:1] + dp_ref[1, :, 0:1] + 1.0
        dinv = lax.rsqrt(deg)
        dinv_ref[...] = dinv
        y1_ref[...] = xw_ref[...] * dinv

    return pl.pallas_call(
        body,
        grid=(NP // BM,),
        in_specs=[
            pl.BlockSpec((2, BM, 16), lambda i: (0, i, 0)),
            pl.BlockSpec((BM, D), lambda i: (i, 0)),
        ],
        out_specs=[
            pl.BlockSpec((BM, D), lambda i: (i, 0)),
            pl.BlockSpec((BM, 1), lambda i: (i, 0)),
        ],
        out_shape=[
            jax.ShapeDtypeStruct((NP, D), jnp.float32),
            jax.ShapeDtypeStruct((NP, 1), jnp.float32),
        ],
    )(dpart, xw)


def _tc_mid(p, y1, dinv, b1, w2p, BM):
    NP, D = y1.shape
    D2 = w2p.shape[1]

    def body(p_ref, y1_ref, dinv_ref, b1_ref, w2_ref, y2_ref):
        agg = p_ref[0] + p_ref[1] + y1_ref[...]
        h = jnp.maximum(dinv_ref[...] * agg + b1_ref[...], 0.0)
        z = jnp.dot(h, w2_ref[...], preferred_element_type=jnp.float32)
        y2_ref[...] = dinv_ref[...] * z

    return pl.pallas_call(
        body,
        grid=(NP // BM,),
        in_specs=[
            pl.BlockSpec((2, BM, D), lambda i: (0, i, 0)),
            pl.BlockSpec((BM, D), lambda i: (i, 0)),
            pl.BlockSpec((BM, 1), lambda i: (i, 0)),
            pl.BlockSpec((1, D), lambda i: (0, 0)),
            pl.BlockSpec((D, D2), lambda i: (0, 0)),
        ],
        out_specs=pl.BlockSpec((BM, D2), lambda i: (i, 0)),
        out_shape=jax.ShapeDtypeStruct((NP, D2), jnp.float32),
    )(p, y1, dinv, b1, w2p)


def _tc_final(q, y2, dinv, b2p, n_valid, BM):
    NP, D2 = y2.shape

    def body(q_ref, y2_ref, dinv_ref, b2_ref, o_ref):
        o = dinv_ref[...] * (q_ref[0] + q_ref[1] + y2_ref[...]) + b2_ref[...]
        col = lax.broadcasted_iota(jnp.int32, (BM, D2), 1)
        valid = col < n_valid
        neg = jnp.where(valid, o, -jnp.inf)
        m = jnp.max(neg, axis=1, keepdims=True)
        e = jnp.where(valid, jnp.exp(o - m), 0.0)
        lse = m + jnp.log(jnp.sum(e, axis=1, keepdims=True))
        o_ref[...] = o - lse

    return pl.pallas_call(
        body,
        grid=(NP // BM,),
        in_specs=[
            pl.BlockSpec((2, BM, D2), lambda i: (0, i, 0)),
            pl.BlockSpec((BM, D2), lambda i: (i, 0)),
            pl.BlockSpec((BM, 1), lambda i: (i, 0)),
            pl.BlockSpec((1, D2), lambda i: (0, 0)),
        ],
        out_specs=pl.BlockSpec((BM, D2), lambda i: (i, 0)),
        out_shape=jax.ShapeDtypeStruct((NP, D2), jnp.float32),
    )(q, y2, dinv, b2p)


def kernel(x, edge_index, W1, b1, W2, b2):
    N, D_IN = x.shape
    E = edge_index.shape[1]
    D_HID = W1.shape[1]
    D_OUT = W2.shape[1]
    D2 = 16

    BM = 1280
    NP = ((N + 1 + BM - 1) // BM) * BM
    JUNK = N

    W = NC * NS
    K = (E + W * CH - 1) // (W * CH)
    K = ((K + 15) // 16) * 16
    EP = W * K * CH

    src = edge_index[0]
    dst = edge_index[1]
    pad = JUNK + (jnp.arange(EP - E, dtype=jnp.int32) % (NP - N))
    src_r = jnp.concatenate([src.astype(jnp.int32), pad]).reshape(W, K, CH)
    dst_r = jnp.concatenate([dst.astype(jnp.int32), pad]).reshape(W, K, CH)
    x_pad = jnp.zeros((NP, D_IN), jnp.float32).at[:N].set(x)
    zeros_hid = jnp.zeros((NP, D_HID), jnp.float32)
    zeros_16 = jnp.zeros((NP, D2), jnp.float32)
    ones_blk = jnp.ones((CH, D2), jnp.float32)
    w2p = jnp.zeros((D_HID, D2), jnp.float32).at[:, :D_OUT].set(W2)
    b1r = b1.reshape(1, D_HID)
    b2p = jnp.zeros((1, D2), jnp.float32).at[0, :D_OUT].set(b2)

    dpart = _make_deg(NP, K)(dst_r, ones_blk, zeros_16)
    xw = _tc_matmul(x_pad, W1, BM)

    y1, dinv = _tc_norm_scale(dpart, xw, BM)

    p = _make_edge_agg(NP, K, D_HID, jnp.float32)(src_r, dst_r, y1, zeros_hid)

    y2 = _tc_mid(p, y1, dinv, b1r, w2p, BM)

    q = _make_edge_agg(NP, K, D2, jnp.float32)(src_r, dst_r, y2, zeros_16)

    o = _tc_final(q, y2, dinv, b2p, D_OUT, BM)
    return o[:N, :D_OUT]


# speedup vs baseline: 2.4252x; 1.0244x over previous
import functools

import jax
import jax.numpy as jnp
from jax import lax
from jax.experimental import pallas as pl
from jax.experimental.pallas import tpu as pltpu
from jax.experimental.pallas import tpu_sc as plsc

NC = 2
NS = 16
CH = 128

_SC_PARAMS = pltpu.CompilerParams(use_tc_tiling_on_sc=False)


def _sc_mesh():
    return plsc.VectorSubcoreMesh(
        core_axis_name="c", subcore_axis_name="s", num_cores=NC, num_subcores=NS
    )


def _make_edge_agg(NP, K, D, table_dtype):
    rows = NP // NS
    assert rows % CH == 0
    H = 2 if D >= 128 else 1
    NB = 2 if D >= 128 else 8
    assert K % (2 * H) == 0 and K % NB == 0
    K2 = K // H

    @functools.partial(
        pl.kernel,
        out_type=jax.ShapeDtypeStruct((NC, NP, D), table_dtype),
        mesh=_sc_mesh(),
        compiler_params=_SC_PARAMS,
        scratch_types=[
            pltpu.VMEM((K2, CH), jnp.int32),
            pltpu.VMEM((K2, CH), jnp.int32),
            pltpu.VMEM((NB, CH, D), table_dtype),
            pltpu.VMEM_SHARED((NP, D), table_dtype),
            pltpu.SemaphoreType.DMA((NB,)),
            pltpu.SemaphoreType.DMA((NB,)),
        ],
    )
    def agg(src_hbm, dst_hbm, table_hbm, zblk_hbm, out_hbm, idx_s, idx_d, msg,
            acc, gsem, ssem):
        c = lax.axis_index("c")
        s = lax.axis_index("s")
        wid = c * NS + s
        pltpu.sync_copy(zblk_hbm, msg.at[0])
        for r in range(rows // CH):
            pltpu.sync_copy(msg.at[0], acc.at[pl.ds(s * rows + r * CH, CH)])
        plsc.subcore_barrier()

        if NB == 2:
            for h in range(H):
                pltpu.sync_copy(src_hbm.at[wid, pl.ds(h * K2, K2)], idx_s)
                pltpu.sync_copy(dst_hbm.at[wid, pl.ds(h * K2, K2)], idx_d)

                pltpu.async_copy(table_hbm.at[idx_s.at[0]], msg.at[0],
                                 gsem.at[0])

                @pl.loop(0, K2, step=2)
                def _(j):
                    for b in range(2):
                        jj = j + b
                        pltpu.make_async_copy(
                            table_hbm.at[idx_s.at[jj]], msg.at[b],
                            gsem.at[b]).wait()
                        nb = 1 - b

                        @pl.when(jj + 1 < K2)
                        def _():
                            pltpu.async_copy(
                                table_hbm.at[idx_s.at[jj + 1]], msg.at[nb],
                                gsem.at[nb])

                        pltpu.sync_copy(msg.at[b], acc.at[idx_d.at[jj]],
                                        add=True)
        else:
            pltpu.sync_copy(src_hbm.at[wid], idx_s)
            pltpu.sync_copy(dst_hbm.at[wid], idx_d)

            @pl.loop(0, K, step=NB)
            def _(j):
                for b in range(NB):
                    pltpu.async_copy(table_hbm.at[idx_s.at[j + b]], msg.at[b],
                                     gsem.at[b])
                for b in range(NB):
                    pltpu.make_async_copy(
                        table_hbm.at[idx_s.at[j + b]], msg.at[b],
                        gsem.at[b]).wait()
                    pltpu.async_copy(msg.at[b], acc.at[idx_d.at[j + b]],
                                     ssem.at[b], add=True)
                for b in range(NB):
                    pltpu.make_async_copy(
                        msg.at[b], acc.at[idx_d.at[j + b]],
                        ssem.at[b]).wait()

        plsc.subcore_barrier()
        pltpu.sync_copy(acc.at[pl.ds(s * rows, rows)],
                        out_hbm.at[c, pl.ds(s * rows, rows)])

    return agg


def _make_deg(NP, K):
    D = 16
    rows = NP // NS

    @functools.partial(
        pl.kernel,
        out_type=jax.ShapeDtypeStruct((NC, NP, D), jnp.float32),
        mesh=_sc_mesh(),
        compiler_params=_SC_PARAMS,
        scratch_types=[
            pltpu.VMEM((K, CH), jnp.int32),
            pltpu.VMEM((CH, D), jnp.float32),
            pltpu.VMEM((CH, D), jnp.float32),
            pltpu.VMEM_SHARED((NP, D), jnp.float32),
            pltpu.SemaphoreType.DMA,
        ],
    )
    def deg(dst_hbm, ones_hbm, zblk_hbm, out_hbm, idx_d, ones_v, zv, acc, ssem):
        c = lax.axis_index("c")
        s = lax.axis_index("s")
        wid = c * NS + s
        pltpu.sync_copy(dst_hbm.at[wid], idx_d)
        pltpu.sync_copy(ones_hbm, ones_v)
        pltpu.sync_copy(zblk_hbm, zv)
        for r in range(rows // CH):
            pltpu.sync_copy(zv, acc.at[pl.ds(s * rows + r * CH, CH)])
        plsc.subcore_barrier()

        @pl.loop(0, K)
        def _(j):
            pltpu.async_copy(ones_v, acc.at[idx_d.at[j]], ssem, add=True)

        @pl.loop(0, K)
        def _(j):
            pltpu.make_async_copy(ones_v, acc.at[idx_d.at[j]], ssem).wait()

        plsc.subcore_barrier()
        pltpu.sync_copy(acc.at[pl.ds(s * rows, rows)],
                        out_hbm.at[c, pl.ds(s * rows, rows)])

    return deg


def _tc_norm_scale(dpart, x, w1, BM):
    NP, Din = x.shape
    D = w1.shape[1]

    def body(dp_ref, x_ref, w_ref, y1_ref, dinv_ref):
        xw = jnp.dot(x_ref[...], w_ref[...], preferred_element_type=jnp.float32)
        deg = dp_ref[0, :, 0:1] + dp_ref[1, :, 0:1] + 1.0
        dinv = lax.rsqrt(deg)
        dinv_ref[...] = dinv
        y1_ref[...] = xw * dinv

    return pl.pallas_call(
        body,
        grid=(NP // BM,),
        in_specs=[
            pl.BlockSpec((2, BM, 16), lambda i: (0, i, 0)),
            pl.BlockSpec((BM, Din), lambda i: (i, 0)),
            pl.BlockSpec((Din, D), lambda i: (0, 0)),
        ],
        out_specs=[
            pl.BlockSpec((BM, D), lambda i: (i, 0)),
            pl.BlockSpec((BM, 1), lambda i: (i, 0)),
        ],
        out_shape=[
            jax.ShapeDtypeStruct((NP, D), jnp.float32),
            jax.ShapeDtypeStruct((NP, 1), jnp.float32),
        ],
    )(dpart, x, w1)


def _tc_mid(p, y1, dinv, b1, w2p, BM):
    NP, D = y1.shape
    D2 = w2p.shape[1]

    def body(p_ref, y1_ref, dinv_ref, b1_ref, w2_ref, y2_ref):
        agg = p_ref[0] + p_ref[1] + y1_ref[...]
        h = jnp.maximum(dinv_ref[...] * agg + b1_ref[...], 0.0)
        z = jnp.dot(h, w2_ref[...], preferred_element_type=jnp.float32)
        y2_ref[...] = dinv_ref[...] * z

    return pl.pallas_call(
        body,
        grid=(NP // BM,),
        in_specs=[
            pl.BlockSpec((2, BM, D), lambda i: (0, i, 0)),
            pl.BlockSpec((BM, D), lambda i: (i, 0)),
            pl.BlockSpec((BM, 1), lambda i: (i, 0)),
            pl.BlockSpec((1, D), lambda i: (0, 0)),
            pl.BlockSpec((D, D2), lambda i: (0, 0)),
        ],
        out_specs=pl.BlockSpec((BM, D2), lambda i: (i, 0)),
        out_shape=jax.ShapeDtypeStruct((NP, D2), jnp.float32),
    )(p, y1, dinv, b1, w2p)


def _tc_final(q, y2, dinv, b2p, n_valid, BM):
    NP, D2 = y2.shape

    def body(q_ref, y2_ref, dinv_ref, b2_ref, o_ref):
        o = dinv_ref[...] * (q_ref[0] + q_ref[1] + y2_ref[...]) + b2_ref[...]
        col = lax.broadcasted_iota(jnp.int32, (BM, D2), 1)
        valid = col < n_valid
        neg = jnp.where(valid, o, -jnp.inf)
        m = jnp.max(neg, axis=1, keepdims=True)
        e = jnp.where(valid, jnp.exp(o - m), 0.0)
        lse = m + jnp.log(jnp.sum(e, axis=1, keepdims=True))
        o_ref[...] = o - lse

    return pl.pallas_call(
        body,
        grid=(NP // BM,),
        in_specs=[
            pl.BlockSpec((2, BM, D2), lambda i: (0, i, 0)),
            pl.BlockSpec((BM, D2), lambda i: (i, 0)),
            pl.BlockSpec((BM, 1), lambda i: (i, 0)),
            pl.BlockSpec((1, D2), lambda i: (0, 0)),
        ],
        out_specs=pl.BlockSpec((BM, D2), lambda i: (i, 0)),
        out_shape=jax.ShapeDtypeStruct((NP, D2), jnp.float32),
    )(q, y2, dinv, b2p)


def kernel(x, edge_index, W1, b1, W2, b2):
    N, D_IN = x.shape
    E = edge_index.shape[1]
    D_HID = W1.shape[1]
    D_OUT = W2.shape[1]
    D2 = 16

    BM = 2560
    NP = ((N + 1 + BM - 1) // BM) * BM
    JUNK = N

    W = NC * NS
    K = (E + W * CH - 1) // (W * CH)
    K = ((K + 15) // 16) * 16
    EP = W * K * CH

    src = edge_index[0]
    dst = edge_index[1]
    pad = JUNK + (jnp.arange(EP - E, dtype=jnp.int32) % (NP - N))
    src_r = jnp.concatenate([src.astype(jnp.int32), pad]).reshape(W, K, CH)
    dst_r = jnp.concatenate([dst.astype(jnp.int32), pad]).reshape(W, K, CH)
    x_pad = jnp.zeros((NP, D_IN), jnp.float32).at[:N].set(x)
    zblk_hid = jnp.zeros((CH, D_HID), jnp.float32)
    zblk_16 = jnp.zeros((CH, D2), jnp.float32)
    ones_blk = jnp.ones((CH, D2), jnp.float32)
    w2p = jnp.zeros((D_HID, D2), jnp.float32).at[:, :D_OUT].set(W2)
    b1r = b1.reshape(1, D_HID)
    b2p = jnp.zeros((1, D2), jnp.float32).at[0, :D_OUT].set(b2)

    dpart = _make_deg(NP, K)(dst_r, ones_blk, zblk_16)

    y1, dinv = _tc_norm_scale(dpart, x_pad, W1, BM)

    p = _make_edge_agg(NP, K, D_HID, jnp.float32)(src_r, dst_r, y1, zblk_hid)

    y2 = _tc_mid(p, y1, dinv, b1r, w2p, BM)

    q = _make_edge_agg(NP, K, D2, jnp.float32)(src_r, dst_r, y2, zblk_16)

    o = _tc_final(q, y2, dinv, b2p, D_OUT, BM)
    return o[:N, :D_OUT]
